# baseline (speedup 1.0000x reference)
import functools

import jax
import jax.numpy as jnp
from jax import lax
from jax.experimental import pallas as pl
from jax.experimental.pallas import tpu as pltpu
from jax.experimental.pallas import tpu_sc as plsc

_NC = 2
_NS = 16
_NW = _NC * _NS
_DP = 128
_BLK = 128

_I16 = None


def _iota16():
    return jax.lax.iota(jnp.int32, 16)


@functools.lru_cache(maxsize=None)
def _make_prep(V: int, D: int):
    n_tblocks = (V + _BLK - 1) // _BLK
    scr_rows = n_tblocks * (_BLK // 4)
    per_w = (n_tblocks + _NW - 1) // _NW
    mesh = plsc.VectorSubcoreMesh(core_axis_name="c", subcore_axis_name="s")

    @functools.partial(
        pl.kernel,
        mesh=mesh,
        out_type=jax.ShapeDtypeStruct((scr_rows, _DP), jnp.float32),
        scratch_types=[
            pltpu.VMEM((2, D, _BLK), jnp.float32),
            pltpu.VMEM((2, _BLK // 4, _DP), jnp.float32),
            [pltpu.SemaphoreType.DMA] * 2,
            [pltpu.SemaphoreType.DMA] * 2,
        ],
        compiler_params=pltpu.CompilerParams(use_tc_tiling_on_sc=True, needs_layout_passes=False),
    )
    def prep(wT_hbm, scr_hbm, slab_v, pad_v, rsems, wsems):
        wid = lax.axis_index("s") * _NC + lax.axis_index("c")
        last = n_tblocks - 1

        def blk_of(c):
            return lax.min(wid * per_w + c, last)

        def start_read(c, b):
            it = blk_of(c)
            return pltpu.async_copy(
                wT_hbm.at[:, pl.ds(it * _BLK, _BLK)], slab_v.at[b], rsems[b])

        def wait_read(c, b):
            it = blk_of(c)
            pltpu.make_async_copy(
                wT_hbm.at[:, pl.ds(it * _BLK, _BLK)], slab_v.at[b],
                rsems[b]).wait()

        def transpose(c, b):
            def tr_body(s_local, _):
                for c0 in range(0, _DP, 16):
                    v = plsc.load_gather(
                        slab_v.at[b],
                        [_iota16() + (c0 % D),
                         jnp.full((16,), 4 * s_local + c0 // D, jnp.int32)])
                    pad_v[b, s_local, pl.ds(c0, 16)] = v
                return _
            lax.fori_loop(0, _BLK // 4, tr_body, 0)

        def start_write(c, b):
            it = blk_of(c)
            return pltpu.async_copy(
                pad_v.at[b], scr_hbm.at[pl.ds(it * (_BLK // 4), _BLK // 4), :],
                wsems[b])

        def wait_write(c, b):
            it = blk_of(c)
            pltpu.make_async_copy(
                pad_v.at[b], scr_hbm.at[pl.ds(it * (_BLK // 4), _BLK // 4), :],
                wsems[b]).wait()

        start_read(0, 0)
        start_read(1, 1)
        for c in (0, 1):
            wait_read(c, c % 2)
            transpose(c, c % 2)
            start_write(c, c % 2)
            start_read(c + 2, c % 2)

        def body(s, _):
            for off in (0, 1):
                c = 2 * s + 2 + off
                wait_read(c, off)
                wait_write(c - 2, off)
                transpose(c, off)
                start_write(c, off)
                start_read(c + 2, off)
            return _

        lax.fori_loop(0, (per_w - 5) // 2, body, 0)

        for c in (per_w - 3, per_w - 2, per_w - 1):
            wait_read(c, c % 2)
            wait_write(c - 2, c % 2)
            transpose(c, c % 2)
            start_write(c, c % 2)
            if c == per_w - 3:
                start_read(c + 2, c % 2)
        wait_write(per_w - 2, (per_w - 2) % 2)
        wait_write(per_w - 1, (per_w - 1) % 2)

    return prep


@functools.lru_cache(maxsize=None)
def _make_lookup(B: int, scr_rows: int, D: int):
    b_per_w = B // _NW
    n_blocks = b_per_w // _BLK
    nbuf = 4
    mesh = plsc.VectorSubcoreMesh(core_axis_name="c", subcore_axis_name="s")

    @functools.partial(
        pl.kernel,
        mesh=mesh,
        out_type=jax.ShapeDtypeStruct((D, B), jnp.float32),
        scratch_types=[
            pltpu.VMEM((b_per_w,), jnp.int32),
            pltpu.VMEM((nbuf, _BLK), jnp.int32),
            pltpu.VMEM((nbuf, _BLK, _DP), jnp.float32),
            pltpu.VMEM((nbuf, D, _BLK), jnp.float32),
            [pltpu.SemaphoreType.DMA] * nbuf,
            [pltpu.SemaphoreType.DMA] * nbuf,
        ],
        compiler_params=pltpu.CompilerParams(use_tc_tiling_on_sc=True, needs_layout_passes=False),
    )
    def lookup(idx_hbm, scr_hbm, outT_hbm, idx_v, idx4_v, rows_v, tb_v,
               gsems, ssems):
        wid = lax.axis_index("s") * _NC + lax.axis_index("c")
        base = wid * b_per_w
        pltpu.sync_copy(idx_hbm.at[pl.ds(base, b_per_w)], idx_v)

        def start_gather(c, b):
            for k0 in range(0, _BLK, 16):
                v = idx_v[pl.ds(c * _BLK + k0, 16)]
                idx4_v[b, pl.ds(k0, 16)] = v >> 2
            return pltpu.async_copy(
                scr_hbm.at[idx4_v.at[b]], rows_v.at[b], gsems[b])

        def wait_gather(c, b):
            pltpu.make_async_copy(
                scr_hbm.at[idx4_v.at[b]], rows_v.at[b], gsems[b]).wait()

        def extract(c, b):
            def ex_body(k8, _):
                k0 = k8 * 16
                rem32 = (idx_v[pl.ds(c * _BLK + k0, 16)] & 3) * D
                rows = _iota16() + k0
                for j in range(D):
                    v = plsc.load_gather(rows_v.at[b], [rows, rem32 + j])
                    tb_v[b, j, pl.ds(k0, 16)] = v
                return _
            lax.fori_loop(0, _BLK // 16, ex_body, 0)

        def start_write(c, b):
            return pltpu.async_copy(
                tb_v.at[b], outT_hbm.at[:, pl.ds(base + c * _BLK, _BLK)],
                ssems[b])

        def wait_write(c, b):
            pltpu.make_async_copy(
                tb_v.at[b], outT_hbm.at[:, pl.ds(base + c * _BLK, _BLK)],
                ssems[b]).wait()

        for c in range(nbuf):
            start_gather(c, c)
        for c in range(nbuf):
            wait_gather(c, c)
            extract(c, c)
            start_write(c, c)
            start_gather(c + nbuf, c)

        def body(s, _):
            for off in range(nbuf):
                c = nbuf * s + nbuf + off
                wait_gather(c, off)
                wait_write(c - nbuf, off)
                extract(c, off)
                start_write(c, off)
                start_gather(c + nbuf, off)
            return _

        lax.fori_loop(0, n_blocks // nbuf - 2, body, 0)

        for c in range(n_blocks - nbuf, n_blocks):
            wait_gather(c, c % nbuf)
            wait_write(c - nbuf, c % nbuf)
            extract(c, c % nbuf)
            start_write(c, c % nbuf)
        for c in range(n_blocks - nbuf, n_blocks):
            wait_write(c, c % nbuf)

    return lookup


def kernel(x, embedding_weights):
    flat = x.reshape(-1).astype(jnp.int32)
    B = flat.shape[0]
    V, D = embedding_weights.shape
    assert D == 32 and B % (_NW * _BLK) == 0, (V, D, B)
    wT = embedding_weights.T
    scratch = _make_prep(V, D)(wT)
    outT = _make_lookup(B, scratch.shape[0], D)(flat, scratch)
    return outT.T


# speedup vs baseline: 4.2201x; 4.2201x over previous
import functools

import jax
import jax.numpy as jnp
from jax import lax
from jax.experimental import pallas as pl
from jax.experimental.pallas import tpu as pltpu
from jax.experimental.pallas import tpu_sc as plsc

_NC = 2
_NS = 16
_NW = _NC * _NS
_DP = 128
_BLK = 128

_I16 = None


def _iota16():
    return jax.lax.iota(jnp.int32, 16)


@functools.lru_cache(maxsize=None)
def _make_prep(V: int, D: int):
    n_tblocks = (V + _BLK - 1) // _BLK
    scr_rows = n_tblocks * (_BLK // 4)
    per_w = (n_tblocks + _NW - 1) // _NW
    mesh = plsc.VectorSubcoreMesh(core_axis_name="c", subcore_axis_name="s")

    @functools.partial(
        pl.kernel,
        mesh=mesh,
        out_type=jax.ShapeDtypeStruct((scr_rows, _DP), jnp.float32),
        scratch_types=[
            pltpu.VMEM((2, D, _BLK), jnp.float32),
            pltpu.VMEM((2, _BLK // 4, _DP), jnp.float32),
            [pltpu.SemaphoreType.DMA] * 2,
            [pltpu.SemaphoreType.DMA] * 2,
        ],
        compiler_params=pltpu.CompilerParams(use_tc_tiling_on_sc=True, needs_layout_passes=False),
    )
    def prep(wT_hbm, scr_hbm, slab_v, pad_v, rsems, wsems):
        wid = lax.axis_index("s") * _NC + lax.axis_index("c")
        last = n_tblocks - 1

        def blk_of(c):
            return lax.min(wid * per_w + c, last)

        def start_read(c, b):
            it = blk_of(c)
            return pltpu.async_copy(
                wT_hbm.at[:, pl.ds(it * _BLK, _BLK)], slab_v.at[b], rsems[b])

        def wait_read(c, b):
            it = blk_of(c)
            pltpu.make_async_copy(
                wT_hbm.at[:, pl.ds(it * _BLK, _BLK)], slab_v.at[b],
                rsems[b]).wait()

        def transpose(c, b):
            rows_lo = _iota16()
            rows_hi = _iota16() + 16

            @functools.partial(plsc.parallel_loop, 0, _BLK // 4, unroll=4)
            def tr_body(s_local):
                s4 = 4 * s_local
                for c0 in range(0, _DP, 16):
                    v = plsc.load_gather(
                        slab_v.at[b],
                        [rows_lo if c0 % D == 0 else rows_hi,
                         jnp.full((16,), s4 + c0 // D, jnp.int32)])
                    pad_v[b, s_local, pl.ds(c0, 16)] = v

        def start_write(c, b):
            it = blk_of(c)
            return pltpu.async_copy(
                pad_v.at[b], scr_hbm.at[pl.ds(it * (_BLK // 4), _BLK // 4), :],
                wsems[b])

        def wait_write(c, b):
            it = blk_of(c)
            pltpu.make_async_copy(
                pad_v.at[b], scr_hbm.at[pl.ds(it * (_BLK // 4), _BLK // 4), :],
                wsems[b]).wait()

        start_read(0, 0)
        start_read(1, 1)
        for c in (0, 1):
            wait_read(c, c % 2)
            transpose(c, c % 2)
            start_write(c, c % 2)
            start_read(c + 2, c % 2)

        def body(s, _):
            for off in (0, 1):
                c = 2 * s + 2 + off
                wait_read(c, off)
                wait_write(c - 2, off)
                transpose(c, off)
                start_write(c, off)
                start_read(c + 2, off)
            return _

        lax.fori_loop(0, (per_w - 5) // 2, body, 0)

        for c in (per_w - 3, per_w - 2, per_w - 1):
            wait_read(c, c % 2)
            wait_write(c - 2, c % 2)
            transpose(c, c % 2)
            start_write(c, c % 2)
            if c == per_w - 3:
                start_read(c + 2, c % 2)
        wait_write(per_w - 2, (per_w - 2) % 2)
        wait_write(per_w - 1, (per_w - 1) % 2)

    return prep


@functools.lru_cache(maxsize=None)
def _make_lookup(B: int, scr_rows: int, D: int):
    b_per_w = B // _NW
    n_blocks = b_per_w // _BLK
    nbuf = 4
    mesh = plsc.VectorSubcoreMesh(core_axis_name="c", subcore_axis_name="s")

    @functools.partial(
        pl.kernel,
        mesh=mesh,
        out_type=jax.ShapeDtypeStruct((D, B), jnp.float32),
        scratch_types=[
            pltpu.VMEM((b_per_w,), jnp.int32),
            pltpu.VMEM((nbuf, _BLK), jnp.int32),
            pltpu.VMEM((nbuf, _BLK, _DP), jnp.float32),
            pltpu.VMEM((nbuf, D, _BLK), jnp.float32),
            [pltpu.SemaphoreType.DMA] * nbuf,
            [pltpu.SemaphoreType.DMA] * nbuf,
        ],
        compiler_params=pltpu.CompilerParams(use_tc_tiling_on_sc=True, needs_layout_passes=False),
    )
    def lookup(idx_hbm, scr_hbm, outT_hbm, idx_v, idx4_v, rows_v, tb_v,
               gsems, ssems):
        wid = lax.axis_index("s") * _NC + lax.axis_index("c")
        base = wid * b_per_w
        pltpu.sync_copy(idx_hbm.at[pl.ds(base, b_per_w)], idx_v)

        def start_gather(c, b):
            for k0 in range(0, _BLK, 16):
                v = idx_v[pl.ds(c * _BLK + k0, 16)]
                idx4_v[b, pl.ds(k0, 16)] = v >> 2
            return pltpu.async_copy(
                scr_hbm.at[idx4_v.at[b]], rows_v.at[b], gsems[b])

        def wait_gather(c, b):
            pltpu.make_async_copy(
                scr_hbm.at[idx4_v.at[b]], rows_v.at[b], gsems[b]).wait()

        def extract(c, b):
            cb = c * _BLK
            ii = _iota16()

            @functools.partial(plsc.parallel_loop, 0, _BLK // 16, unroll=2)
            def ex_body(k8):
                k0 = k8 * 16
                rem32 = (idx_v[pl.ds(cb + k0, 16)] & 3) * D
                rows = ii + k0
                for j in range(D):
                    v = plsc.load_gather(rows_v.at[b], [rows, rem32 + j])
                    tb_v[b, j, pl.ds(k0, 16)] = v

        def start_write(c, b):
            return pltpu.async_copy(
                tb_v.at[b], outT_hbm.at[:, pl.ds(base + c * _BLK, _BLK)],
                ssems[b])

        def wait_write(c, b):
            pltpu.make_async_copy(
                tb_v.at[b], outT_hbm.at[:, pl.ds(base + c * _BLK, _BLK)],
                ssems[b]).wait()

        for c in range(nbuf):
            start_gather(c, c)
        for c in range(nbuf):
            wait_gather(c, c)
            extract(c, c)
            start_write(c, c)
            start_gather(c + nbuf, c)

        def body(s, _):
            for off in range(nbuf):
                c = nbuf * s + nbuf + off
                wait_gather(c, off)
                wait_write(c - nbuf, off)
                extract(c, off)
                start_write(c, off)
                start_gather(c + nbuf, off)
            return _

        lax.fori_loop(0, n_blocks // nbuf - 2, body, 0)

        for c in range(n_blocks - nbuf, n_blocks):
            wait_gather(c, c % nbuf)
            wait_write(c - nbuf, c % nbuf)
            extract(c, c % nbuf)
            start_write(c, c % nbuf)
        for c in range(n_blocks - nbuf, n_blocks):
            wait_write(c, c % nbuf)

    return lookup


def kernel(x, embedding_weights):
    flat = x.reshape(-1).astype(jnp.int32)
    B = flat.shape[0]
    V, D = embedding_weights.shape
    assert D == 32 and B % (_NW * _BLK) == 0, (V, D, B)
    wT = embedding_weights.T
    scratch = _make_prep(V, D)(wT)
    outT = _make_lookup(B, scratch.shape[0], D)(flat, scratch)
    return outT.T
